# Initial kernel scaffold; baseline (speedup 1.0000x reference)
import jax
import jax.numpy as jnp
from jax.experimental import pallas as pl


def kernel(x, edge_index, edge_weight, batch, W1, b1, W2, b2, W3, b3):
    raise NotImplementedError("write your pallas kernel here")


# speedup vs baseline: 6.8495x; 6.8495x over previous
import jax
import jax.numpy as jnp
from jax import lax
from jax.experimental import pallas as pl
from jax.experimental.pallas import tpu as pltpu
from jax.experimental.pallas import tpu_sc as plsc

N_NODES_K = 10000
N_PAD = 10240
N_EDGES_K = 160000
E_PAD = 163840
N_TILES = 16
N_BLOCKS = 80
BLK = 128
CW = 128
N_HALF = N_PAD // 2
ACC_REAL = N_HALF
ACC_ROWS = N_HALF + BLK
STRIPE = N_PAD // N_TILES
F32 = jnp.float32
I32 = jnp.int32

_SC_PARAMS = pltpu.CompilerParams(needs_layout_passes=False)


def _rsqrt16(v):
    i = lax.bitcast_convert_type(v, I32)
    y = lax.bitcast_convert_type(jnp.int32(0x5F3759DF) - (i >> 1), F32)
    for _ in range(3):
        y = y * (1.5 - 0.5 * v * y * y)
    return y


def _sc_kernel_n(row_h, col_h, ew_h, norm_h, dis_h,
                 row_v, col_v, ew_v, deg_v, dis_v, iota_v, tmp_v, stripe_v,
                 deg_sh, dis_sh):
    c = lax.axis_index("c")
    s = lax.axis_index("s")
    off = s * STRIPE
    srow = s * (N_BLOCKS // N_TILES)

    pltpu.sync_copy(row_h.at[s], row_v)
    pltpu.sync_copy(col_h.at[s], col_v)
    pltpu.sync_copy(ew_h.at[s], ew_v)

    z = jnp.zeros((16,), F32)

    def zdeg(i, _):
        deg_v[i // 8, pl.ds((i % 8) * 16, 16)] = z
        return _

    lax.fori_loop(0, N_PAD // 16, zdeg, None)

    pltpu.sync_copy(deg_v.at[pl.ds(0, 5)], deg_sh.at[pl.ds(srow, 5)])
    for i in range(5):
        iota_v[pl.ds(i * 16, 16)] = lax.iota(I32, 16) + (i * 16)

    def dg(i, _):
        blk = i // 8
        j = (i % 8) * 16
        idx = col_v[blk, pl.ds(j, 16)]
        w = ew_v[blk, pl.ds(j, 16)]
        plsc.addupdate_scatter(deg_v, [idx >> 7, idx & 127], w)
        return _

    plsc.subcore_barrier()
    lax.fori_loop(0, N_BLOCKS * 8, dg, None)

    pltpu.sync_copy(deg_v, deg_sh.at[iota_v], add=True)
    plsc.subcore_barrier()

    pltpu.sync_copy(deg_sh.at[pl.ds(srow, 5)], tmp_v)

    def rs(i, _):
        a = tmp_v[i // 8, pl.ds((i % 8) * 16, 16)] + 1.0
        stripe_v[pl.ds(i * 16, 16)] = _rsqrt16(a)
        return _

    lax.fori_loop(0, STRIPE // 16, rs, None)

    pltpu.sync_copy(stripe_v, dis_sh.at[pl.ds(off, STRIPE)])

    @pl.when(c == 0)
    def _():
        pltpu.sync_copy(stripe_v, dis_h.at[pl.ds(off, STRIPE)])

    plsc.subcore_barrier()
    pltpu.sync_copy(dis_sh, dis_v)

    def nm(i, _):
        blk = i // 8
        j = (i % 8) * 16
        r = row_v[blk, pl.ds(j, 16)]
        cc = col_v[blk, pl.ds(j, 16)]
        w = ew_v[blk, pl.ds(j, 16)]
        nrm = plsc.load_gather(dis_v, [r]) * w * plsc.load_gather(dis_v, [cc])
        ew_v[blk, pl.ds(j, 16)] = nrm
        return _

    lax.fori_loop(0, N_BLOCKS * 8, nm, None)

    @pl.when(c == 0)
    def _():
        pltpu.sync_copy(ew_v, norm_h.at[s])


def _sc_kernel_g(row_h, col_h, nrm_h, src_stk, out_stk,
                 row_v, col_v, nrm_v, brow, bcol, bnrm,
                 gbuf, zbuf, acc_sh, sem):
    c = lax.axis_index("c")
    s = lax.axis_index("s")

    pltpu.sync_copy(row_h.at[s], row_v)
    pltpu.sync_copy(col_h.at[s], col_v)
    pltpu.sync_copy(nrm_h.at[s], nrm_v)

    z = jnp.zeros((16,), F32)

    def zz(i, _):
        zbuf[i // 8, pl.ds((i % 8) * 16, 16)] = z
        return _

    lax.fori_loop(0, (8 * BLK) // 16, zz, None)

    fill_col = jnp.full((16,), ACC_REAL + 8 * s, I32) + (lax.iota(I32, 16) & 7)

    zoff = s * (ACC_ROWS // N_TILES)
    doff = s * (N_HALF // N_TILES)

    for h in range(2):
        lo = h * N_HALF

        def pf(i, _):
            blk = i // 8
            j = (i % 8) * 16
            rfill = i * 16 + lax.iota(I32, 16)
            rfill = jnp.where(rfill < N_PAD, rfill, rfill - N_PAD)
            brow[blk, pl.ds(j, 16)] = rfill
            bcol[blk, pl.ds(j, 16)] = fill_col
            bnrm[blk, pl.ds(j, 16)] = z
            return _

        lax.fori_loop(0, N_BLOCKS * 8, pf, None)

        def bn(i, off):
            blk = i // 8
            j = (i % 8) * 16
            r = row_v[blk, pl.ds(j, 16)]
            cg = col_v[blk, pl.ds(j, 16)]
            w = nrm_v[blk, pl.ds(j, 16)]
            cl = cg - lo
            m = (cl >= 0) & (cl < N_HALF)
            pref = plsc.cumsum(m.astype(I32))
            pos = off + pref - 1
            plsc.store_scatter(brow, [pos >> 7, pos & 127], r, mask=m)
            plsc.store_scatter(bcol, [pos >> 7, pos & 127], cl, mask=m)
            plsc.store_scatter(bnrm, [pos >> 7, pos & 127], w, mask=m)
            return off + pref[15]

        off_e = lax.fori_loop(0, N_BLOCKS * 8, bn, jnp.int32(0))
        nblk = (off_e + BLK - 1) >> 7

        for q in range(2):
            kidx = c * 2 + q

            def za(i, _):
                pltpu.sync_copy(zbuf, acc_sh.at[pl.ds(zoff + i * 8, 8)])
                return _

            lax.fori_loop(0, ACC_ROWS // N_TILES // 8, za, None)
            plsc.subcore_barrier()

            src = src_stk.at[kidx]

            def blkbody(b, _):
                pltpu.async_copy(src.at[brow.at[b]], gbuf, sem).wait()

                def mul(jj, _2):
                    nv = bnrm[b, pl.ds(jj * 16, 16)]
                    for l in range(16):
                        sv = jnp.full((16,), nv[l], F32)
                        j = jj * 16 + l
                        for g in range(CW // 16):
                            sl = pl.ds(g * 16, 16)
                            gbuf[j, sl] = gbuf[j, sl] * sv
                    return _2

                lax.fori_loop(0, BLK // 16, mul, None)
                pltpu.sync_copy(gbuf, acc_sh.at[bcol.at[b]], add=True)
                return _

            lax.fori_loop(0, nblk, blkbody, None)
            plsc.subcore_barrier()
            pltpu.sync_copy(
                acc_sh.at[pl.ds(doff, N_HALF // N_TILES)],
                out_stk.at[kidx, pl.ds(h * N_HALF + doff,
                                       N_HALF // N_TILES)])
            plsc.subcore_barrier()


def _make_sc_n():
    mesh = plsc.VectorSubcoreMesh(core_axis_name="c", subcore_axis_name="s")
    out_type = (
        jax.ShapeDtypeStruct((N_TILES, N_BLOCKS, BLK), F32),
        jax.ShapeDtypeStruct((N_PAD,), F32))
    scratch = [
        pltpu.VMEM((N_BLOCKS, BLK), I32),
        pltpu.VMEM((N_BLOCKS, BLK), I32),
        pltpu.VMEM((N_BLOCKS, BLK), F32),
        pltpu.VMEM((N_BLOCKS, BLK), F32),
        pltpu.VMEM((N_PAD,), F32),
        pltpu.VMEM((N_BLOCKS,), I32),
        pltpu.VMEM((5, BLK), F32),
        pltpu.VMEM((STRIPE,), F32),
        pltpu.VMEM_SHARED((N_BLOCKS, BLK), F32),
        pltpu.VMEM_SHARED((N_PAD,), F32),
    ]

    def fn(*args):
        return pl.kernel(_sc_kernel_n, out_type=out_type, mesh=mesh,
                         scratch_types=scratch,
                         compiler_params=_SC_PARAMS)(*args)

    return fn


def _make_sc_g():
    mesh = plsc.VectorSubcoreMesh(core_axis_name="c", subcore_axis_name="s")
    out_type = jax.ShapeDtypeStruct((4, N_PAD, CW), F32)
    scratch = [
        pltpu.VMEM((N_BLOCKS, BLK), I32),
        pltpu.VMEM((N_BLOCKS, BLK), I32),
        pltpu.VMEM((N_BLOCKS, BLK), F32),
        pltpu.VMEM((N_BLOCKS, BLK), I32),
        pltpu.VMEM((N_BLOCKS, BLK), I32),
        pltpu.VMEM((N_BLOCKS, BLK), F32),
        pltpu.VMEM((BLK, CW), F32),
        pltpu.VMEM((8, BLK), F32),
        pltpu.VMEM_SHARED((ACC_ROWS, CW), F32),
        pltpu.SemaphoreType.DMA,
    ]

    def fn(*args):
        return pl.kernel(_sc_kernel_g, out_type=out_type, mesh=mesh,
                         scratch_types=scratch,
                         compiler_params=_SC_PARAMS)(*args)

    return fn


def _tc_matmul_stk(x_ref, w_ref, out_ref):
    h = jnp.dot(x_ref[...], w_ref[...], preferred_element_type=F32)
    out_ref[...] = h.reshape(BLK, 4, CW).transpose(1, 0, 2)


def _tc_kernel_mid(agg_ref, hw_ref, dis_ref, b1_ref, w2_ref, out_ref):
    dd = dis_ref[0, 0, :]
    sn = (dd * dd)[:, None]
    a = agg_ref[...]
    hw = hw_ref[...]
    agg = jnp.concatenate([a[0], a[1], a[2], a[3]], axis=1)
    hwc = jnp.concatenate([hw[0], hw[1], hw[2], hw[3]], axis=1)
    h1 = jnp.maximum(agg + sn * hwc + b1_ref[...], 0.0)
    h = jnp.dot(h1, w2_ref[...], preferred_element_type=F32)
    out_ref[...] = h.reshape(BLK, 4, CW).transpose(1, 0, 2)


def _tc_kernel_post(agg_ref, hw2_ref, dis_ref, batch_ref,
                    b2_ref, w3_ref, b3_ref, out_ref, g_acc, cnt):
    i = pl.program_id(0)

    @pl.when(i == 0)
    def _():
        g_acc[...] = jnp.zeros_like(g_acc)
        cnt[...] = jnp.zeros_like(cnt)

    dd = dis_ref[0, 0, :]
    sn = (dd * dd)[:, None]
    a = agg_ref[...]
    hw = hw2_ref[...]
    agg = jnp.concatenate([a[0], a[1], a[2], a[3]], axis=1)
    hwc = jnp.concatenate([hw[0], hw[1], hw[2], hw[3]], axis=1)
    h2 = jnp.maximum(agg + sn * hwc + b2_ref[...], 0.0)

    bb = batch_ref[0, 0, :]
    iota = lax.broadcasted_iota(I32, (64, BLK), 0)
    onehot = (bb[None, :] == iota).astype(F32)
    g_acc[...] += jnp.dot(onehot, h2, preferred_element_type=F32)
    cnt[...] += jnp.dot(onehot, jnp.ones((BLK, 128), F32),
                        preferred_element_type=F32)

    @pl.when(i == N_BLOCKS - 1)
    def _():
        counts = jnp.maximum(cnt[:, 0:1], 1.0)
        g = g_acc[...] / counts
        out_ref[...] = (jnp.dot(g, w3_ref[...], preferred_element_type=F32)
                        + b3_ref[...])


def kernel(x, edge_index, edge_weight, batch, W1, b1, W2, b2, W3, b3):
    row = edge_index[0].astype(I32)
    col = edge_index[1].astype(I32)
    ew = edge_weight.astype(F32)

    npad_e = E_PAD - N_EDGES_K
    pad_ids = jnp.arange(npad_e, dtype=I32)
    row_p = jnp.concatenate([row, pad_ids % N_PAD])
    col_p = jnp.concatenate([col, N_NODES_K + pad_ids % (N_PAD - N_NODES_K)])
    ew_p = jnp.concatenate([ew, jnp.zeros((npad_e,), F32)])
    row3 = row_p.reshape(N_TILES, N_BLOCKS, BLK)
    col3 = col_p.reshape(N_TILES, N_BLOCKS, BLK)
    ew3 = ew_p.reshape(N_TILES, N_BLOCKS, BLK)

    x_pad = jnp.concatenate(
        [x, jnp.zeros((N_PAD - N_NODES_K, x.shape[1]), F32)])

    batch_p = jnp.concatenate(
        [batch.astype(I32), jnp.full((N_PAD - N_NODES_K,), 64, I32)])
    batch3 = batch_p.reshape(N_BLOCKS, 1, BLK)

    norm3, dis = _make_sc_n()(row3, col3, ew3)
    dis3 = dis.reshape(N_BLOCKS, 1, BLK)

    sc_g = _make_sc_g()

    hw = pl.pallas_call(
        _tc_matmul_stk,
        grid=(N_BLOCKS,),
        in_specs=[
            pl.BlockSpec((BLK, 256), lambda i: (i, 0)),
            pl.BlockSpec((256, 512), lambda i: (0, 0)),
        ],
        out_specs=pl.BlockSpec((4, BLK, CW), lambda i: (0, i, 0)),
        out_shape=jax.ShapeDtypeStruct((4, N_PAD, CW), F32),
    )(x_pad, W1)

    agg1 = sc_g(row3, col3, norm3, hw)

    hw2 = pl.pallas_call(
        _tc_kernel_mid,
        grid=(N_BLOCKS,),
        in_specs=[
            pl.BlockSpec((4, BLK, CW), lambda i: (0, i, 0)),
            pl.BlockSpec((4, BLK, CW), lambda i: (0, i, 0)),
            pl.BlockSpec((1, 1, BLK), lambda i: (i, 0, 0)),
            pl.BlockSpec((1, 512), lambda i: (0, 0)),
            pl.BlockSpec((512, 512), lambda i: (0, 0)),
        ],
        out_specs=pl.BlockSpec((4, BLK, CW), lambda i: (0, i, 0)),
        out_shape=jax.ShapeDtypeStruct((4, N_PAD, CW), F32),
    )(agg1, hw, dis3, b1.reshape(1, 512), W2)

    agg2 = sc_g(row3, col3, norm3, hw2)

    out = pl.pallas_call(
        _tc_kernel_post,
        grid=(N_BLOCKS,),
        in_specs=[
            pl.BlockSpec((4, BLK, CW), lambda i: (0, i, 0)),
            pl.BlockSpec((4, BLK, CW), lambda i: (0, i, 0)),
            pl.BlockSpec((1, 1, BLK), lambda i: (i, 0, 0)),
            pl.BlockSpec((1, 1, BLK), lambda i: (i, 0, 0)),
            pl.BlockSpec((1, 512), lambda i: (0, 0)),
            pl.BlockSpec((512, 128), lambda i: (0, 0)),
            pl.BlockSpec((1, 128), lambda i: (0, 0)),
        ],
        out_specs=pl.BlockSpec((64, 128), lambda i: (0, 0)),
        out_shape=jax.ShapeDtypeStruct((64, 128), F32),
        scratch_shapes=[
            pltpu.VMEM((64, 512), F32),
            pltpu.VMEM((64, 128), F32),
        ],
    )(agg2, hw2, dis3, batch3, b2.reshape(1, 512),
      W3, b3.reshape(1, 128))

    return out


# speedup vs baseline: 9.0139x; 1.3160x over previous
import jax
import jax.numpy as jnp
from jax import lax
from jax.experimental import pallas as pl
from jax.experimental.pallas import tpu as pltpu
from jax.experimental.pallas import tpu_sc as plsc

N_NODES_K = 10000
N_PAD = 10240
N_EDGES_K = 160000
E_PAD = 163840
N_TILES = 16
N_BLOCKS = 80
BLK = 128
CW = 128
N_Q = N_PAD // 4
ACC_ROWS = N_Q + 64
STRIPE = N_PAD // N_TILES
F32 = jnp.float32
I32 = jnp.int32

_SC_PARAMS = pltpu.CompilerParams(needs_layout_passes=False)


def _rsqrt16(v):
    i = lax.bitcast_convert_type(v, I32)
    y = lax.bitcast_convert_type(jnp.int32(0x5F3759DF) - (i >> 1), F32)
    for _ in range(3):
        y = y * (1.5 - 0.5 * v * y * y)
    return y


def _sc_kernel_n(row_h, col_h, ew_h, norm_h, dis_h,
                 row_v, col_v, ew_v, deg_v, dis_v, iota_v, tmp_v, stripe_v,
                 deg_sh, dis_sh):
    c = lax.axis_index("c")
    s = lax.axis_index("s")
    off = s * STRIPE
    srow = s * (N_BLOCKS // N_TILES)

    pltpu.sync_copy(row_h.at[s], row_v)
    pltpu.sync_copy(col_h.at[s], col_v)
    pltpu.sync_copy(ew_h.at[s], ew_v)

    z = jnp.zeros((16,), F32)

    def zdeg(i, _):
        deg_v[i // 8, pl.ds((i % 8) * 16, 16)] = z
        return _

    lax.fori_loop(0, N_PAD // 16, zdeg, None)

    pltpu.sync_copy(deg_v.at[pl.ds(0, 5)], deg_sh.at[pl.ds(srow, 5)])
    for i in range(5):
        iota_v[pl.ds(i * 16, 16)] = lax.iota(I32, 16) + (i * 16)

    def dg(i, _):
        blk = i // 8
        j = (i % 8) * 16
        idx = col_v[blk, pl.ds(j, 16)]
        w = ew_v[blk, pl.ds(j, 16)]
        plsc.addupdate_scatter(deg_v, [idx >> 7, idx & 127], w)
        return _

    plsc.subcore_barrier()
    lax.fori_loop(0, N_BLOCKS * 8, dg, None)

    pltpu.sync_copy(deg_v, deg_sh.at[iota_v], add=True)
    plsc.subcore_barrier()

    pltpu.sync_copy(deg_sh.at[pl.ds(srow, 5)], tmp_v)

    def rs(i, _):
        a = tmp_v[i // 8, pl.ds((i % 8) * 16, 16)] + 1.0
        stripe_v[pl.ds(i * 16, 16)] = _rsqrt16(a)
        return _

    lax.fori_loop(0, STRIPE // 16, rs, None)

    pltpu.sync_copy(stripe_v, dis_sh.at[pl.ds(off, STRIPE)])

    @pl.when(c == 0)
    def _():
        pltpu.sync_copy(stripe_v, dis_h.at[pl.ds(off, STRIPE)])

    plsc.subcore_barrier()
    pltpu.sync_copy(dis_sh, dis_v)

    def nm(i, _):
        blk = i // 8
        j = (i % 8) * 16
        r = row_v[blk, pl.ds(j, 16)]
        cc = col_v[blk, pl.ds(j, 16)]
        w = ew_v[blk, pl.ds(j, 16)]
        nrm = plsc.load_gather(dis_v, [r]) * w * plsc.load_gather(dis_v, [cc])
        ew_v[blk, pl.ds(j, 16)] = nrm
        return _

    lax.fori_loop(0, N_BLOCKS * 8, nm, None)

    @pl.when(c == 0)
    def _():
        pltpu.sync_copy(ew_v, norm_h.at[s])


def _sc_kernel_g(row_h, col_h, nrm_h, src_stk, out_stk,
                 row_v, col_v, nrm_v, brow, bcol, bnrm,
                 gbuf0, gbuf1, zbuf, acc_sh, gsem0, gsem1, ssem0, ssem1):
    c = lax.axis_index("c")
    s = lax.axis_index("s")

    pltpu.sync_copy(row_h.at[s], row_v)
    pltpu.sync_copy(col_h.at[s], col_v)
    pltpu.sync_copy(nrm_h.at[s], nrm_v)

    z = jnp.zeros((16,), F32)

    def zz(i, _):
        zbuf[i // 8, pl.ds((i % 8) * 16, 16)] = z
        return _

    lax.fori_loop(0, (8 * BLK) // 16, zz, None)

    fill_col = jnp.full((16,), N_Q + 4 * s, I32) + (lax.iota(I32, 16) & 3)

    zoff = s * (ACC_ROWS // N_TILES)
    doff = s * (N_Q // N_TILES)

    def scale(buf, b):
        def mul(jj, _2):
            nv = bnrm[b, pl.ds(jj * 16, 16)]
            for l in range(16):
                sv = jnp.full((16,), nv[l], F32)
                j = jj * 16 + l
                for g in range(CW // 16):
                    sl = pl.ds(g * 16, 16)
                    buf[j, sl] = buf[j, sl] * sv
            return _2

        lax.fori_loop(0, BLK // 16, mul, None)

    for qt in range(4):
        lo = qt * N_Q

        def bn(i, off):
            blk = i // 8
            j = (i % 8) * 16
            r = row_v[blk, pl.ds(j, 16)]
            cg = col_v[blk, pl.ds(j, 16)]
            w = nrm_v[blk, pl.ds(j, 16)]
            cl = cg - lo
            m = (cl >= 0) & (cl < N_Q)
            pref = plsc.cumsum(m.astype(I32))
            pos = off + pref - 1
            plsc.store_scatter(brow, [pos >> 7, pos & 127], r, mask=m)
            plsc.store_scatter(bcol, [pos >> 7, pos & 127], cl, mask=m)
            plsc.store_scatter(bnrm, [pos >> 7, pos & 127], w, mask=m)
            return off + pref[15]

        off_e = lax.fori_loop(0, N_BLOCKS * 8, bn, jnp.int32(0))
        nblk = (off_e + BLK - 1) >> 7

        @pl.when(off_e > 0)
        def _():
            lastb = nblk - 1
            rel = off_e - lastb * BLK

            def tf(j, _):
                sl = pl.ds(j * 16, 16)
                idx = j * 16 + lax.iota(I32, 16)
                m = idx >= rel
                rfill = s * BLK + idx
                brow[lastb, sl] = jnp.where(m, rfill, brow[lastb, sl])
                bcol[lastb, sl] = jnp.where(m, fill_col, bcol[lastb, sl])
                bnrm[lastb, sl] = jnp.where(m, z, bnrm[lastb, sl])
                return _

            lax.fori_loop(0, 8, tf, None)

        for q in range(2):
            kidx = c * 2 + q

            def za(i, _):
                pltpu.sync_copy(zbuf, acc_sh.at[pl.ds(zoff + i * 8, 8)])
                return _

            lax.fori_loop(0, 20, za, None)
            pltpu.sync_copy(zbuf.at[pl.ds(0, 4)],
                            acc_sh.at[pl.ds(zoff + 160, 4)])
            plsc.subcore_barrier()

            src = src_stk.at[kidx]

            @pl.when(nblk > 0)
            def _():
                pltpu.async_copy(src.at[brow.at[0]], gbuf0, gsem0)

            def pairbody(p, _):
                b0 = p * 2
                b1 = b0 + 1
                b2 = b0 + 2

                @pl.when(b1 < nblk)
                def _():
                    @pl.when(b1 >= 3)
                    def _():
                        pltpu.make_async_copy(
                            gbuf1, acc_sh.at[bcol.at[0]], ssem1).wait()

                    pltpu.async_copy(src.at[brow.at[b1]], gbuf1, gsem1)

                pltpu.make_async_copy(src.at[brow.at[b0]], gbuf0,
                                      gsem0).wait()
                scale(gbuf0, b0)
                pltpu.async_copy(gbuf0, acc_sh.at[bcol.at[b0]], ssem0,
                                 add=True)

                @pl.when(b2 < nblk)
                def _():
                    pltpu.make_async_copy(
                        gbuf0, acc_sh.at[bcol.at[0]], ssem0).wait()
                    pltpu.async_copy(src.at[brow.at[b2]], gbuf0, gsem0)

                @pl.when(b1 < nblk)
                def _():
                    pltpu.make_async_copy(src.at[brow.at[b1]], gbuf1,
                                          gsem1).wait()
                    scale(gbuf1, b1)
                    pltpu.async_copy(gbuf1, acc_sh.at[bcol.at[b1]], ssem1,
                                     add=True)

                return _

            lax.fori_loop(0, (nblk + 1) >> 1, pairbody, None)

            @pl.when(nblk > 0)
            def _():
                pltpu.make_async_copy(gbuf0, acc_sh.at[bcol.at[0]],
                                      ssem0).wait()

            @pl.when(nblk > 1)
            def _():
                pltpu.make_async_copy(gbuf1, acc_sh.at[bcol.at[0]],
                                      ssem1).wait()

            plsc.subcore_barrier()
            pltpu.sync_copy(
                acc_sh.at[pl.ds(doff, N_Q // N_TILES)],
                out_stk.at[kidx, pl.ds(lo + doff, N_Q // N_TILES)])
            plsc.subcore_barrier()


def _make_sc_n():
    mesh = plsc.VectorSubcoreMesh(core_axis_name="c", subcore_axis_name="s")
    out_type = (
        jax.ShapeDtypeStruct((N_TILES, N_BLOCKS, BLK), F32),
        jax.ShapeDtypeStruct((N_PAD,), F32))
    scratch = [
        pltpu.VMEM((N_BLOCKS, BLK), I32),
        pltpu.VMEM((N_BLOCKS, BLK), I32),
        pltpu.VMEM((N_BLOCKS, BLK), F32),
        pltpu.VMEM((N_BLOCKS, BLK), F32),
        pltpu.VMEM((N_PAD,), F32),
        pltpu.VMEM((N_BLOCKS,), I32),
        pltpu.VMEM((5, BLK), F32),
        pltpu.VMEM((STRIPE,), F32),
        pltpu.VMEM_SHARED((N_BLOCKS, BLK), F32),
        pltpu.VMEM_SHARED((N_PAD,), F32),
    ]

    def fn(*args):
        return pl.kernel(_sc_kernel_n, out_type=out_type, mesh=mesh,
                         scratch_types=scratch,
                         compiler_params=_SC_PARAMS)(*args)

    return fn


def _make_sc_g():
    mesh = plsc.VectorSubcoreMesh(core_axis_name="c", subcore_axis_name="s")
    out_type = jax.ShapeDtypeStruct((4, N_PAD, CW), F32)
    scratch = [
        pltpu.VMEM((N_BLOCKS, BLK), I32),
        pltpu.VMEM((N_BLOCKS, BLK), I32),
        pltpu.VMEM((N_BLOCKS, BLK), F32),
        pltpu.VMEM((N_BLOCKS, BLK), I32),
        pltpu.VMEM((N_BLOCKS, BLK), I32),
        pltpu.VMEM((N_BLOCKS, BLK), F32),
        pltpu.VMEM((BLK, CW), F32),
        pltpu.VMEM((BLK, CW), F32),
        pltpu.VMEM((8, BLK), F32),
        pltpu.VMEM_SHARED((ACC_ROWS, CW), F32),
        pltpu.SemaphoreType.DMA,
        pltpu.SemaphoreType.DMA,
        pltpu.SemaphoreType.DMA,
        pltpu.SemaphoreType.DMA,
    ]

    def fn(*args):
        return pl.kernel(_sc_kernel_g, out_type=out_type, mesh=mesh,
                         scratch_types=scratch,
                         compiler_params=_SC_PARAMS)(*args)

    return fn


def _tc_matmul_stk(x_ref, w_ref, out_ref):
    h = jnp.dot(x_ref[...], w_ref[...], preferred_element_type=F32)
    out_ref[...] = h.reshape(BLK, 4, CW).transpose(1, 0, 2)


def _tc_kernel_mid(agg_ref, hw_ref, dis_ref, b1_ref, w2_ref, out_ref):
    dd = dis_ref[0, 0, :]
    sn = (dd * dd)[:, None]
    a = agg_ref[...]
    hw = hw_ref[...]
    agg = jnp.concatenate([a[0], a[1], a[2], a[3]], axis=1)
    hwc = jnp.concatenate([hw[0], hw[1], hw[2], hw[3]], axis=1)
    h1 = jnp.maximum(agg + sn * hwc + b1_ref[...], 0.0)
    h = jnp.dot(h1, w2_ref[...], preferred_element_type=F32)
    out_ref[...] = h.reshape(BLK, 4, CW).transpose(1, 0, 2)


def _tc_kernel_post(agg_ref, hw2_ref, dis_ref, batch_ref,
                    b2_ref, w3_ref, b3_ref, out_ref, g_acc, cnt):
    i = pl.program_id(0)

    @pl.when(i == 0)
    def _():
        g_acc[...] = jnp.zeros_like(g_acc)
        cnt[...] = jnp.zeros_like(cnt)

    dd = dis_ref[0, 0, :]
    sn = (dd * dd)[:, None]
    a = agg_ref[...]
    hw = hw2_ref[...]
    agg = jnp.concatenate([a[0], a[1], a[2], a[3]], axis=1)
    hwc = jnp.concatenate([hw[0], hw[1], hw[2], hw[3]], axis=1)
    h2 = jnp.maximum(agg + sn * hwc + b2_ref[...], 0.0)

    bb = batch_ref[0, 0, :]
    iota = lax.broadcasted_iota(I32, (64, BLK), 0)
    onehot = (bb[None, :] == iota).astype(F32)
    g_acc[...] += jnp.dot(onehot, h2, preferred_element_type=F32)
    cnt[...] += jnp.dot(onehot, jnp.ones((BLK, 128), F32),
                        preferred_element_type=F32)

    @pl.when(i == N_BLOCKS - 1)
    def _():
        counts = jnp.maximum(cnt[:, 0:1], 1.0)
        g = g_acc[...] / counts
        out_ref[...] = (jnp.dot(g, w3_ref[...], preferred_element_type=F32)
                        + b3_ref[...])


def kernel(x, edge_index, edge_weight, batch, W1, b1, W2, b2, W3, b3):
    row = edge_index[0].astype(I32)
    col = edge_index[1].astype(I32)
    ew = edge_weight.astype(F32)

    npad_e = E_PAD - N_EDGES_K
    pad_ids = jnp.arange(npad_e, dtype=I32)
    row_p = jnp.concatenate([row, pad_ids % N_PAD])
    col_p = jnp.concatenate([col, N_NODES_K + pad_ids % (N_PAD - N_NODES_K)])
    ew_p = jnp.concatenate([ew, jnp.zeros((npad_e,), F32)])
    row3 = row_p.reshape(N_TILES, N_BLOCKS, BLK)
    col3 = col_p.reshape(N_TILES, N_BLOCKS, BLK)
    ew3 = ew_p.reshape(N_TILES, N_BLOCKS, BLK)

    x_pad = jnp.concatenate(
        [x, jnp.zeros((N_PAD - N_NODES_K, x.shape[1]), F32)])

    batch_p = jnp.concatenate(
        [batch.astype(I32), jnp.full((N_PAD - N_NODES_K,), 64, I32)])
    batch3 = batch_p.reshape(N_BLOCKS, 1, BLK)

    norm3, dis = _make_sc_n()(row3, col3, ew3)
    dis3 = dis.reshape(N_BLOCKS, 1, BLK)

    sc_g = _make_sc_g()

    hw = pl.pallas_call(
        _tc_matmul_stk,
        grid=(N_BLOCKS,),
        in_specs=[
            pl.BlockSpec((BLK, 256), lambda i: (i, 0)),
            pl.BlockSpec((256, 512), lambda i: (0, 0)),
        ],
        out_specs=pl.BlockSpec((4, BLK, CW), lambda i: (0, i, 0)),
        out_shape=jax.ShapeDtypeStruct((4, N_PAD, CW), F32),
    )(x_pad, W1)

    agg1 = sc_g(row3, col3, norm3, hw)

    hw2 = pl.pallas_call(
        _tc_kernel_mid,
        grid=(N_BLOCKS,),
        in_specs=[
            pl.BlockSpec((4, BLK, CW), lambda i: (0, i, 0)),
            pl.BlockSpec((4, BLK, CW), lambda i: (0, i, 0)),
            pl.BlockSpec((1, 1, BLK), lambda i: (i, 0, 0)),
            pl.BlockSpec((1, 512), lambda i: (0, 0)),
            pl.BlockSpec((512, 512), lambda i: (0, 0)),
        ],
        out_specs=pl.BlockSpec((4, BLK, CW), lambda i: (0, i, 0)),
        out_shape=jax.ShapeDtypeStruct((4, N_PAD, CW), F32),
    )(agg1, hw, dis3, b1.reshape(1, 512), W2)

    agg2 = sc_g(row3, col3, norm3, hw2)

    out = pl.pallas_call(
        _tc_kernel_post,
        grid=(N_BLOCKS,),
        in_specs=[
            pl.BlockSpec((4, BLK, CW), lambda i: (0, i, 0)),
            pl.BlockSpec((4, BLK, CW), lambda i: (0, i, 0)),
            pl.BlockSpec((1, 1, BLK), lambda i: (i, 0, 0)),
            pl.BlockSpec((1, 1, BLK), lambda i: (i, 0, 0)),
            pl.BlockSpec((1, 512), lambda i: (0, 0)),
            pl.BlockSpec((512, 128), lambda i: (0, 0)),
            pl.BlockSpec((1, 128), lambda i: (0, 0)),
        ],
        out_specs=pl.BlockSpec((64, 128), lambda i: (0, 0)),
        out_shape=jax.ShapeDtypeStruct((64, 128), F32),
        scratch_shapes=[
            pltpu.VMEM((64, 512), F32),
            pltpu.VMEM((64, 128), F32),
        ],
    )(agg2, hw2, dis3, batch3, b2.reshape(1, 512),
      W3, b3.reshape(1, 128))

    return out


# speedup vs baseline: 9.0766x; 1.0070x over previous
import jax
import jax.numpy as jnp
from jax import lax
from jax.experimental import pallas as pl
from jax.experimental.pallas import tpu as pltpu
from jax.experimental.pallas import tpu_sc as plsc

N_NODES_K = 10000
N_PAD = 10240
N_EDGES_K = 160000
E_PAD = 163840
N_TILES = 16
N_BLOCKS = 80
BLK = 128
CW = 128
N_P = 2048
STRIPE = N_PAD // N_TILES
F32 = jnp.float32
I32 = jnp.int32

_SC_PARAMS = pltpu.CompilerParams(needs_layout_passes=False)


def _rsqrt16(v):
    i = lax.bitcast_convert_type(v, I32)
    y = lax.bitcast_convert_type(jnp.int32(0x5F3759DF) - (i >> 1), F32)
    for _ in range(3):
        y = y * (1.5 - 0.5 * v * y * y)
    return y


def _sc_kernel_n(row_h, col_h, ew_h, norm_h, dis_h,
                 row_v, col_v, ew_v, deg_v, dis_v, iota_v, tmp_v, stripe_v,
                 deg_sh, dis_sh):
    c = lax.axis_index("c")
    s = lax.axis_index("s")
    off = s * STRIPE
    srow = s * (N_BLOCKS // N_TILES)

    pltpu.sync_copy(row_h.at[s], row_v)
    pltpu.sync_copy(col_h.at[s], col_v)
    pltpu.sync_copy(ew_h.at[s], ew_v)

    z = jnp.zeros((16,), F32)

    def zdeg(i, _):
        deg_v[i // 8, pl.ds((i % 8) * 16, 16)] = z
        return _

    lax.fori_loop(0, N_PAD // 16, zdeg, None)

    pltpu.sync_copy(deg_v.at[pl.ds(0, 5)], deg_sh.at[pl.ds(srow, 5)])
    for i in range(5):
        iota_v[pl.ds(i * 16, 16)] = lax.iota(I32, 16) + (i * 16)

    def dg(i, _):
        blk = i // 8
        j = (i % 8) * 16
        idx = col_v[blk, pl.ds(j, 16)]
        w = ew_v[blk, pl.ds(j, 16)]
        plsc.addupdate_scatter(deg_v, [idx >> 7, idx & 127], w)
        return _

    plsc.subcore_barrier()
    lax.fori_loop(0, N_BLOCKS * 8, dg, None)

    pltpu.sync_copy(deg_v, deg_sh.at[iota_v], add=True)
    plsc.subcore_barrier()

    pltpu.sync_copy(deg_sh.at[pl.ds(srow, 5)], tmp_v)

    def rs(i, _):
        a = tmp_v[i // 8, pl.ds((i % 8) * 16, 16)] + 1.0
        stripe_v[pl.ds(i * 16, 16)] = _rsqrt16(a)
        return _

    lax.fori_loop(0, STRIPE // 16, rs, None)

    pltpu.sync_copy(stripe_v, dis_sh.at[pl.ds(off, STRIPE)])

    @pl.when(c == 0)
    def _():
        pltpu.sync_copy(stripe_v, dis_h.at[pl.ds(off, STRIPE)])

    plsc.subcore_barrier()
    pltpu.sync_copy(dis_sh, dis_v)

    def nm(i, _):
        blk = i // 8
        j = (i % 8) * 16
        r = row_v[blk, pl.ds(j, 16)]
        cc = col_v[blk, pl.ds(j, 16)]
        w = ew_v[blk, pl.ds(j, 16)]
        nrm = plsc.load_gather(dis_v, [r]) * w * plsc.load_gather(dis_v, [cc])
        ew_v[blk, pl.ds(j, 16)] = nrm
        return _

    lax.fori_loop(0, N_BLOCKS * 8, nm, None)

    @pl.when(c == 0)
    def _():
        pltpu.sync_copy(ew_v, norm_h.at[s])


def _sc_kernel_g(row_h, col_h, nrm_h, src_stk, out_stk,
                 row_v, col_v, nrm_v, brow, bcol, bnrm,
                 gbuf0, gbuf1, gbuf2, acc_sh,
                 gsem0, gsem1, gsem2, ssem0, ssem1, ssem2):
    c = lax.axis_index("c")
    s = lax.axis_index("s")

    pltpu.sync_copy(row_h.at[s], row_v)
    pltpu.sync_copy(col_h.at[s], col_v)
    pltpu.sync_copy(nrm_h.at[s], nrm_v)

    z = jnp.zeros((16,), F32)

    doff = s * (N_P // N_TILES)
    fill_col = jnp.full((16,), doff, I32) + (lax.iota(I32, 16) & 15)

    def scale(buf, b):
        def mul(jj, _2):
            nv = bnrm[b, pl.ds(jj * 16, 16)]
            for l in range(16):
                sv = jnp.full((16,), nv[l], F32)
                j = jj * 16 + l
                for g in range(CW // 16):
                    sl = pl.ds(g * 16, 16)
                    buf[j, sl] = buf[j, sl] * sv
            return _2

        lax.fori_loop(0, BLK // 16, mul, None)

    for qt in range(N_PAD // N_P):
        lo = qt * N_P

        def bn(i, off):
            blk = i // 8
            j = (i % 8) * 16
            r = row_v[blk, pl.ds(j, 16)]
            cg = col_v[blk, pl.ds(j, 16)]
            w = nrm_v[blk, pl.ds(j, 16)]
            cl = cg - lo
            m = (cl >= 0) & (cl < N_P)
            pref = plsc.cumsum(m.astype(I32))
            pos = off + pref - 1
            plsc.store_scatter(brow, [pos >> 7, pos & 127], r, mask=m)
            plsc.store_scatter(bcol, [pos >> 7, pos & 127], cl, mask=m)
            plsc.store_scatter(bnrm, [pos >> 7, pos & 127], w, mask=m)
            return off + pref[15]

        off_e = lax.fori_loop(0, N_BLOCKS * 8, bn, jnp.int32(0))
        nblk = (off_e + BLK - 1) >> 7

        @pl.when(off_e > 0)
        def _():
            lastb = nblk - 1
            rel = off_e - lastb * BLK

            def tf(j, _):
                sl = pl.ds(j * 16, 16)
                idx = j * 16 + lax.iota(I32, 16)
                m = idx >= rel
                rfill = s * BLK + idx
                brow[lastb, sl] = jnp.where(m, rfill, brow[lastb, sl])
                bcol[lastb, sl] = jnp.where(m, fill_col, bcol[lastb, sl])
                bnrm[lastb, sl] = jnp.where(m, z, bnrm[lastb, sl])
                return _

            lax.fori_loop(0, 8, tf, None)

        for q in range(2):
            kidx = c * 2 + q

            def zg(i, _):
                gbuf2[i // 8, pl.ds((i % 8) * 16, 16)] = z
                return _

            lax.fori_loop(0, (8 * CW) // 16, zg, None)

            def za(i, _):
                pltpu.sync_copy(gbuf2.at[pl.ds(0, 8)],
                                acc_sh.at[pl.ds(doff + i * 8, 8)])
                return _

            lax.fori_loop(0, (N_P // N_TILES) // 8, za, None)
            plsc.subcore_barrier()

            src = src_stk.at[kidx]

            def wait_g(buf, gsem, bi):
                pltpu.make_async_copy(src.at[brow.at[bi]], buf, gsem).wait()

            def wait_s(buf, ssem):
                pltpu.make_async_copy(buf, acc_sh.at[bcol.at[0]],
                                      ssem).wait()

            def do_scale(buf, bi, ssem):
                scale(buf, bi)
                pltpu.async_copy(buf, acc_sh.at[bcol.at[bi]], ssem,
                                 add=True)

            @pl.when(nblk > 0)
            def _():
                pltpu.async_copy(src.at[brow.at[0]], gbuf0, gsem0)

            @pl.when(nblk > 1)
            def _():
                pltpu.async_copy(src.at[brow.at[1]], gbuf1, gsem1)

            def ringbody(t, _):
                ba = t * 3
                bb = ba + 1
                bc = ba + 2

                @pl.when(bc < nblk)
                def _():
                    @pl.when(bc > 2)
                    def _():
                        wait_s(gbuf2, ssem2)

                    pltpu.async_copy(src.at[brow.at[bc]], gbuf2, gsem2)

                wait_g(gbuf0, gsem0, ba)
                do_scale(gbuf0, ba, ssem0)

                @pl.when(bb < nblk)
                def _():
                    wait_g(gbuf1, gsem1, bb)
                    do_scale(gbuf1, bb, ssem1)

                @pl.when(ba + 3 < nblk)
                def _():
                    wait_s(gbuf0, ssem0)
                    pltpu.async_copy(src.at[brow.at[ba + 3]], gbuf0, gsem0)

                @pl.when(bc < nblk)
                def _():
                    wait_g(gbuf2, gsem2, bc)
                    do_scale(gbuf2, bc, ssem2)

                @pl.when(bb + 3 < nblk)
                def _():
                    wait_s(gbuf1, ssem1)
                    pltpu.async_copy(src.at[brow.at[bb + 3]], gbuf1, gsem1)

                return _

            lax.fori_loop(0, (nblk + 2) // 3, ringbody, None)

            @pl.when(nblk > 0)
            def _():
                wait_s(gbuf0, ssem0)

            @pl.when(nblk > 1)
            def _():
                wait_s(gbuf1, ssem1)

            @pl.when(nblk > 2)
            def _():
                wait_s(gbuf2, ssem2)

            plsc.subcore_barrier()
            pltpu.sync_copy(
                acc_sh.at[pl.ds(doff, N_P // N_TILES)],
                out_stk.at[kidx, pl.ds(lo + doff, N_P // N_TILES)])
            plsc.subcore_barrier()


def _make_sc_n():
    mesh = plsc.VectorSubcoreMesh(core_axis_name="c", subcore_axis_name="s")
    out_type = (
        jax.ShapeDtypeStruct((N_TILES, N_BLOCKS, BLK), F32),
        jax.ShapeDtypeStruct((N_PAD,), F32))
    scratch = [
        pltpu.VMEM((N_BLOCKS, BLK), I32),
        pltpu.VMEM((N_BLOCKS, BLK), I32),
        pltpu.VMEM((N_BLOCKS, BLK), F32),
        pltpu.VMEM((N_BLOCKS, BLK), F32),
        pltpu.VMEM((N_PAD,), F32),
        pltpu.VMEM((N_BLOCKS,), I32),
        pltpu.VMEM((5, BLK), F32),
        pltpu.VMEM((STRIPE,), F32),
        pltpu.VMEM_SHARED((N_BLOCKS, BLK), F32),
        pltpu.VMEM_SHARED((N_PAD,), F32),
    ]

    def fn(*args):
        return pl.kernel(_sc_kernel_n, out_type=out_type, mesh=mesh,
                         scratch_types=scratch,
                         compiler_params=_SC_PARAMS)(*args)

    return fn


def _make_sc_g():
    mesh = plsc.VectorSubcoreMesh(core_axis_name="c", subcore_axis_name="s")
    out_type = jax.ShapeDtypeStruct((4, N_PAD, CW), F32)
    scratch = [
        pltpu.VMEM((N_BLOCKS, BLK), I32),
        pltpu.VMEM((N_BLOCKS, BLK), I32),
        pltpu.VMEM((N_BLOCKS, BLK), F32),
        pltpu.VMEM((N_BLOCKS, BLK), I32),
        pltpu.VMEM((N_BLOCKS, BLK), I32),
        pltpu.VMEM((N_BLOCKS, BLK), F32),
        pltpu.VMEM((BLK, CW), F32),
        pltpu.VMEM((BLK, CW), F32),
        pltpu.VMEM((BLK, CW), F32),
        pltpu.VMEM_SHARED((N_P, CW), F32),
        pltpu.SemaphoreType.DMA,
        pltpu.SemaphoreType.DMA,
        pltpu.SemaphoreType.DMA,
        pltpu.SemaphoreType.DMA,
        pltpu.SemaphoreType.DMA,
        pltpu.SemaphoreType.DMA,
    ]

    def fn(*args):
        return pl.kernel(_sc_kernel_g, out_type=out_type, mesh=mesh,
                         scratch_types=scratch,
                         compiler_params=_SC_PARAMS)(*args)

    return fn


def _tc_matmul_stk(x_ref, w_ref, out_ref):
    h = jnp.dot(x_ref[...], w_ref[...], preferred_element_type=F32)
    out_ref[...] = h.reshape(BLK, 4, CW).transpose(1, 0, 2)


def _tc_kernel_mid(agg_ref, hw_ref, dis_ref, b1_ref, w2_ref, out_ref):
    dd = dis_ref[0, 0, :]
    sn = (dd * dd)[:, None]
    a = agg_ref[...]
    hw = hw_ref[...]
    agg = jnp.concatenate([a[0], a[1], a[2], a[3]], axis=1)
    hwc = jnp.concatenate([hw[0], hw[1], hw[2], hw[3]], axis=1)
    h1 = jnp.maximum(agg + sn * hwc + b1_ref[...], 0.0)
    h = jnp.dot(h1, w2_ref[...], preferred_element_type=F32)
    out_ref[...] = h.reshape(BLK, 4, CW).transpose(1, 0, 2)


def _tc_kernel_post(agg_ref, hw2_ref, dis_ref, batch_ref,
                    b2_ref, w3_ref, b3_ref, out_ref, g_acc, cnt):
    i = pl.program_id(0)

    @pl.when(i == 0)
    def _():
        g_acc[...] = jnp.zeros_like(g_acc)
        cnt[...] = jnp.zeros_like(cnt)

    dd = dis_ref[0, 0, :]
    sn = (dd * dd)[:, None]
    a = agg_ref[...]
    hw = hw2_ref[...]
    agg = jnp.concatenate([a[0], a[1], a[2], a[3]], axis=1)
    hwc = jnp.concatenate([hw[0], hw[1], hw[2], hw[3]], axis=1)
    h2 = jnp.maximum(agg + sn * hwc + b2_ref[...], 0.0)

    bb = batch_ref[0, 0, :]
    iota = lax.broadcasted_iota(I32, (64, BLK), 0)
    onehot = (bb[None, :] == iota).astype(F32)
    g_acc[...] += jnp.dot(onehot, h2, preferred_element_type=F32)
    cnt[...] += jnp.dot(onehot, jnp.ones((BLK, 128), F32),
                        preferred_element_type=F32)

    @pl.when(i == N_BLOCKS - 1)
    def _():
        counts = jnp.maximum(cnt[:, 0:1], 1.0)
        g = g_acc[...] / counts
        out_ref[...] = (jnp.dot(g, w3_ref[...], preferred_element_type=F32)
                        + b3_ref[...])


def kernel(x, edge_index, edge_weight, batch, W1, b1, W2, b2, W3, b3):
    row = edge_index[0].astype(I32)
    col = edge_index[1].astype(I32)
    ew = edge_weight.astype(F32)

    npad_e = E_PAD - N_EDGES_K
    pad_ids = jnp.arange(npad_e, dtype=I32)
    row_p = jnp.concatenate([row, pad_ids % N_PAD])
    col_p = jnp.concatenate([col, N_NODES_K + pad_ids % (N_PAD - N_NODES_K)])
    ew_p = jnp.concatenate([ew, jnp.zeros((npad_e,), F32)])
    row3 = row_p.reshape(N_TILES, N_BLOCKS, BLK)
    col3 = col_p.reshape(N_TILES, N_BLOCKS, BLK)
    ew3 = ew_p.reshape(N_TILES, N_BLOCKS, BLK)

    x_pad = jnp.concatenate(
        [x, jnp.zeros((N_PAD - N_NODES_K, x.shape[1]), F32)])

    batch_p = jnp.concatenate(
        [batch.astype(I32), jnp.full((N_PAD - N_NODES_K,), 64, I32)])
    batch3 = batch_p.reshape(N_BLOCKS, 1, BLK)

    norm3, dis = _make_sc_n()(row3, col3, ew3)
    dis3 = dis.reshape(N_BLOCKS, 1, BLK)

    sc_g = _make_sc_g()

    hw = pl.pallas_call(
        _tc_matmul_stk,
        grid=(N_BLOCKS,),
        in_specs=[
            pl.BlockSpec((BLK, 256), lambda i: (i, 0)),
            pl.BlockSpec((256, 512), lambda i: (0, 0)),
        ],
        out_specs=pl.BlockSpec((4, BLK, CW), lambda i: (0, i, 0)),
        out_shape=jax.ShapeDtypeStruct((4, N_PAD, CW), F32),
    )(x_pad, W1)

    agg1 = sc_g(row3, col3, norm3, hw)

    hw2 = pl.pallas_call(
        _tc_kernel_mid,
        grid=(N_BLOCKS,),
        in_specs=[
            pl.BlockSpec((4, BLK, CW), lambda i: (0, i, 0)),
            pl.BlockSpec((4, BLK, CW), lambda i: (0, i, 0)),
            pl.BlockSpec((1, 1, BLK), lambda i: (i, 0, 0)),
            pl.BlockSpec((1, 512), lambda i: (0, 0)),
            pl.BlockSpec((512, 512), lambda i: (0, 0)),
        ],
        out_specs=pl.BlockSpec((4, BLK, CW), lambda i: (0, i, 0)),
        out_shape=jax.ShapeDtypeStruct((4, N_PAD, CW), F32),
    )(agg1, hw, dis3, b1.reshape(1, 512), W2)

    agg2 = sc_g(row3, col3, norm3, hw2)

    out = pl.pallas_call(
        _tc_kernel_post,
        grid=(N_BLOCKS,),
        in_specs=[
            pl.BlockSpec((4, BLK, CW), lambda i: (0, i, 0)),
            pl.BlockSpec((4, BLK, CW), lambda i: (0, i, 0)),
            pl.BlockSpec((1, 1, BLK), lambda i: (i, 0, 0)),
            pl.BlockSpec((1, 1, BLK), lambda i: (i, 0, 0)),
            pl.BlockSpec((1, 512), lambda i: (0, 0)),
            pl.BlockSpec((512, 128), lambda i: (0, 0)),
            pl.BlockSpec((1, 128), lambda i: (0, 0)),
        ],
        out_specs=pl.BlockSpec((64, 128), lambda i: (0, 0)),
        out_shape=jax.ShapeDtypeStruct((64, 128), F32),
        scratch_shapes=[
            pltpu.VMEM((64, 512), F32),
            pltpu.VMEM((64, 128), F32),
        ],
    )(agg2, hw2, dis3, batch3, b2.reshape(1, 512),
      W3, b3.reshape(1, 128))

    return out
